# speedup vs baseline: 9.1368x; 9.1368x over previous
import jax
import jax.numpy as jnp
from jax.experimental import pallas as pl
from jax.experimental.pallas import tpu as pltpu
from jax.experimental.pallas import tpu_sc as plsc

VOCAB = 100000
HIDDEN = 128
BATCH = 4096
HIST = 200

GATHER_WINDOW = 256


def _sc_gather(table, flat_tokens):
    num_indices = flat_tokens.shape[0]
    mesh = plsc.VectorSubcoreMesh(core_axis_name="c", subcore_axis_name="s")

    @pl.kernel(
        out_type=jax.ShapeDtypeStruct((num_indices, HIDDEN), jnp.float32),
        mesh=mesh,
    )
    def gather_kernel(table_hbm, idx_hbm, out_hbm):
        def body(idx_vmem, out_vmem):
            pltpu.sync_copy(table_hbm.at[idx_vmem.at[0]], out_vmem)

        pltpu.emit_pipeline(
            body,
            grid=(num_indices // GATHER_WINDOW,),
            in_specs=[
                pl.BlockSpec((1, GATHER_WINDOW), index_map=lambda i: (0, i))
            ],
            out_specs=[
                pl.BlockSpec((GATHER_WINDOW, HIDDEN), index_map=lambda i: (i, 0))
            ],
            core_axis_name=("c", "s"),
            dimension_semantics=(pltpu.PARALLEL,),
        )(idx_hbm, out_hbm)

    return gather_kernel(table, flat_tokens.reshape(1, num_indices))


def kernel(tokens, token_emb):
    flat = tokens.reshape(-1).astype(jnp.int32)
    out = _sc_gather(token_emb, flat)
    return out.reshape(tokens.shape[0], tokens.shape[1], HIDDEN)
